# Initial kernel scaffold; baseline (speedup 1.0000x reference)
import jax
import jax.numpy as jnp
from jax.experimental import pallas as pl


def kernel(h, x, edges, vel, edge_attr, charges, Wemb, bemb, eW1, eb1, eW2, eb2, nW1, nb1, nW2, nb2, cW1, cb1, cW2, vW1, vb1, vW2, vb2, fE, fW1, fb1, fW2, fb2, fW3, fb3):
    raise NotImplementedError("write your pallas kernel here")


# speedup vs baseline: 3.0315x; 3.0315x over previous
import functools

import jax
import jax.numpy as jnp
from jax import lax
from jax.experimental import pallas as pl
from jax.experimental.pallas import tpu as pltpu
from jax.experimental.pallas import tpu_sc as plsc

N = 10000
E = 320000
HID = 128
NC, NS = 2, 16
NWK = NC * NS
EPW = E // NWK
CH = 80
NCHK = EPW // CH
NPT = 624
NREM = N - NS * NPT
TRN = 104
NB = 2000
EBK = 2000

_UNTILED = pltpu.CompilerParams(use_tc_tiling_on_sc=False)


def _silu(v):
    return v * jax.nn.sigmoid(v)


def _full(shape):
    return pl.BlockSpec(shape, lambda i: tuple(0 for _ in shape))


def _blk(bs, w):
    return pl.BlockSpec((bs, w), lambda i: (i, 0))


def _init_body(x, vel, ch, h, fE0, fE1, fW1, fb1, fW2, fb2, fW3, fb3,
               Wemb, bemb, field_o, hh_o):
    fec = jnp.where(ch[...] == 0, fE0[...], fE1[...])
    finp = jnp.concatenate([x[...], vel[...], fec], axis=1)
    t = _silu(jnp.dot(finp, fW1[...], preferred_element_type=jnp.float32) + fb1[...])
    t = _silu(jnp.dot(t, fW2[...], preferred_element_type=jnp.float32) + fb2[...])
    field_o[...] = jnp.dot(t, fW3[...], preferred_element_type=jnp.float32) + fb3[...]
    hh_o[...] = jnp.dot(h[...], Wemb[...], preferred_element_type=jnp.float32) + bemb[...]


def _node_init(x, vel, ch, h, fE0, fE1, fW1, fb1, fW2, fb2, fW3, fb3, Wemb, bemb):
    return pl.pallas_call(
        _init_body,
        grid=(N // NB,),
        in_specs=[_blk(NB, 3), _blk(NB, 3), _blk(NB, 1), _blk(NB, 16),
                  _full((1, 16)), _full((1, 16)),
                  _full((22, 32)), _full((1, 32)), _full((32, 32)), _full((1, 32)),
                  _full((32, 3)), _full((1, 3)),
                  _full((16, HID)), _full((1, HID))],
        out_specs=[_blk(NB, 3), _blk(NB, HID)],
        out_shape=[jax.ShapeDtypeStruct((N, 3), jnp.float32),
                   jax.ShapeDtypeStruct((N, HID), jnp.float32)],
    )(x, vel, ch, h, fE0, fE1, fW1, fb1, fW2, fb2, fW3, fb3, Wemb, bemb)


def _pre_body(hh, coord, Wr, Wc, eb1r, vW1, vb1, vW2, vb2,
              ta_o, tb_o, cp_o, velw_o):
    b = hh.shape[0]
    ta_o[...] = jnp.dot(hh[...], Wr[...], preferred_element_type=jnp.float32) + eb1r[...]
    tb_o[...] = jnp.dot(hh[...], Wc[...], preferred_element_type=jnp.float32)
    cp_o[...] = jnp.concatenate([coord[...], jnp.zeros((b, 13), jnp.float32)], axis=1)
    v1 = _silu(jnp.dot(hh[...], vW1[...], preferred_element_type=jnp.float32) + vb1[...])
    velw_o[...] = jnp.dot(v1, vW2[...], preferred_element_type=jnp.float32) + vb2[...]


def _node_pre(hh, coord, Wr, Wc, eb1r, vW1, vb1, vW2, vb2):
    return pl.pallas_call(
        _pre_body,
        grid=(N // NB,),
        in_specs=[_blk(NB, HID), _blk(NB, 3),
                  _full((HID, HID)), _full((HID, HID)), _full((1, HID)),
                  _full((HID, HID)), _full((1, HID)), _full((HID, 1)), _full((1, 1))],
        out_specs=[_blk(NB, HID), _blk(NB, HID), _blk(NB, 16), _blk(NB, 1)],
        out_shape=[jax.ShapeDtypeStruct((N, HID), jnp.float32),
                   jax.ShapeDtypeStruct((N, HID), jnp.float32),
                   jax.ShapeDtypeStruct((N, 16), jnp.float32),
                   jax.ShapeDtypeStruct((N, 1), jnp.float32)],
    )(hh, coord, Wr, Wc, eb1r, vW1, vb1, vW2, vb2)


def _sc_gather(ta, tb, row3, col3):
    mesh = plsc.VectorSubcoreMesh(core_axis_name="c", subcore_axis_name="s")

    @functools.partial(
        pl.kernel,
        out_type=jax.ShapeDtypeStruct((E, HID), jnp.float32),
        mesh=mesh,
        scratch_types=[
            pltpu.VMEM((NCHK, CH), jnp.int32),
            pltpu.VMEM((NCHK, CH), jnp.int32),
            pltpu.VMEM((CH, HID), jnp.float32),
            pltpu.VMEM((CH, HID), jnp.float32),
            pltpu.VMEM((CH, HID), jnp.float32),
            pltpu.SemaphoreType.DMA,
            pltpu.SemaphoreType.DMA,
        ],
    )
    def k(ta_h, tb_h, row_h, col_h, out_h, ridx, cidx, ba, bb, bs, sa, sb):
        wid = lax.axis_index("c") * NS + lax.axis_index("s")
        pltpu.sync_copy(row_h.at[wid], ridx)
        pltpu.sync_copy(col_h.at[wid], cidx)
        base = wid * EPW

        def chunk(c, carry):
            ca = pltpu.async_copy(ta_h.at[ridx.at[c]], ba, sa)
            cb = pltpu.async_copy(tb_h.at[cidx.at[c]], bb, sb)
            ca.wait()
            cb.wait()

            def addrow(r, carry2):
                for j in range(HID // 16):
                    sl = pl.ds(j * 16, 16)
                    bs[r, sl] = ba[r, sl] + bb[r, sl]
                return carry2

            lax.fori_loop(0, CH, addrow, 0, unroll=False)
            pltpu.sync_copy(bs, out_h.at[pl.ds(base + c * CH, CH)])
            return carry

        lax.fori_loop(0, NCHK, chunk, 0, unroll=False)

    return k(ta, tb, row3, col3)


def _sc_gather_cd(cp, row3, col3):
    mesh = plsc.VectorSubcoreMesh(core_axis_name="c", subcore_axis_name="s")

    @functools.partial(
        pl.kernel,
        out_type=jax.ShapeDtypeStruct((E, 16), jnp.float32),
        mesh=mesh,
        compiler_params=_UNTILED,
        scratch_types=[
            pltpu.VMEM((NCHK, CH), jnp.int32),
            pltpu.VMEM((NCHK, CH), jnp.int32),
            pltpu.VMEM((CH, 16), jnp.float32),
            pltpu.VMEM((CH, 16), jnp.float32),
            pltpu.VMEM((CH, 16), jnp.float32),
            pltpu.SemaphoreType.DMA,
            pltpu.SemaphoreType.DMA,
        ],
    )
    def k(cp_h, row_h, col_h, out_h, ridx, cidx, ba, bb, bs, sa, sb):
        wid = lax.axis_index("c") * NS + lax.axis_index("s")
        pltpu.sync_copy(row_h.at[wid], ridx)
        pltpu.sync_copy(col_h.at[wid], cidx)
        base = wid * EPW

        def chunk(c, carry):
            ca = pltpu.async_copy(cp_h.at[ridx.at[c]], ba, sa)
            cb = pltpu.async_copy(cp_h.at[cidx.at[c]], bb, sb)
            ca.wait()
            cb.wait()

            def subrow(r, carry2):
                bs[r, :] = ba[r, :] - bb[r, :]
                return carry2

            lax.fori_loop(0, CH, subrow, 0, unroll=False)
            pltpu.sync_copy(bs, out_h.at[pl.ds(base + c * CH, CH)])
            return carry

        lax.fori_loop(0, NCHK, chunk, 0, unroll=False)

    return k(cp, row3, col3)


def _edge_body(s, cdr, ea, Wea, wr, eW2, eb2, cW1, cb1, cW2, m_o, t_o):
    cd = cdr[...]
    radial = jnp.sum(cd * cd, axis=1, keepdims=True)
    pre = (s[...]
           + jnp.dot(ea[...], Wea[...], preferred_element_type=jnp.float32)
           + radial * wr[...])
    m1 = _silu(pre)
    m = _silu(jnp.dot(m1, eW2[...], preferred_element_type=jnp.float32) + eb2[...])
    c1 = _silu(jnp.dot(m, cW1[...], preferred_element_type=jnp.float32) + cb1[...])
    p = jnp.dot(c1, cW2[...], preferred_element_type=jnp.float32)
    lane = lax.broadcasted_iota(jnp.int32, (1, 16), 1)
    t_o[...] = cd * p + jnp.where(lane == 3, 1.0, 0.0)
    m_o[...] = m


def _edge_mlp(s, cd, ea, Wea, wr, eW2, eb2, cW1, cb1, cW2):
    return pl.pallas_call(
        _edge_body,
        grid=(E // EBK,),
        in_specs=[_blk(EBK, HID), _blk(EBK, 16), _blk(EBK, 4),
                  _full((4, HID)), _full((1, HID)),
                  _full((HID, HID)), _full((1, HID)),
                  _full((HID, HID)), _full((1, HID)), _full((HID, 1))],
        out_specs=[_blk(EBK, HID), _blk(EBK, 16)],
        out_shape=[jax.ShapeDtypeStruct((E, HID), jnp.float32),
                   jax.ShapeDtypeStruct((E, 16), jnp.float32)],
    )(s, cd, ea, Wea, wr, eW2, eb2, cW1, cb1, cW2)


def _sc_scatter_m(m, row3):
    mesh = plsc.VectorSubcoreMesh(core_axis_name="c", subcore_axis_name="s")

    @functools.partial(
        pl.kernel,
        out_type=jax.ShapeDtypeStruct((NC, N, HID), jnp.float32),
        mesh=mesh,
        scratch_types=[
            pltpu.VMEM((NCHK, CH), jnp.int32),
            pltpu.VMEM((CH, HID), jnp.float32),
            pltpu.VMEM((TRN, HID), jnp.float32),
            pltpu.VMEM_SHARED((N, HID), jnp.float32),
        ],
    )
    def k(m_h, row_h, nag_o, ridx, mb, zb, spn):
        cid = lax.axis_index("c")
        sid = lax.axis_index("s")
        wid = cid * NS + sid

        def zrow(r, carry):
            for j in range(HID // 16):
                zb[r, pl.ds(j * 16, 16)] = jnp.zeros((16,), jnp.float32)
            return carry

        lax.fori_loop(0, TRN, zrow, 0, unroll=False)
        for t in range(NPT // TRN):
            pltpu.sync_copy(zb, spn.at[pl.ds(sid * NPT + t * TRN, TRN)])

        @pl.when(sid == NS - 1)
        def _zrem():
            pltpu.sync_copy(zb.at[pl.ds(0, NREM)],
                            spn.at[pl.ds(NS * NPT, NREM)])

        plsc.subcore_barrier()

        pltpu.sync_copy(row_h.at[wid], ridx)
        base = wid * EPW

        def chunk(c, carry):
            pltpu.sync_copy(m_h.at[pl.ds(base + c * CH, CH)], mb)
            pltpu.sync_copy(mb, spn.at[ridx.at[c]], add=True)
            return carry

        lax.fori_loop(0, NCHK, chunk, 0, unroll=False)
        plsc.subcore_barrier()

        for t in range(NPT // TRN):
            sl = pl.ds(sid * NPT + t * TRN, TRN)
            pltpu.sync_copy(spn.at[sl], zb)
            pltpu.sync_copy(zb, nag_o.at[cid].at[sl])

        @pl.when(sid == NS - 1)
        def _wrem():
            sl = pl.ds(NS * NPT, NREM)
            pltpu.sync_copy(spn.at[sl], zb.at[pl.ds(0, NREM)])
            pltpu.sync_copy(zb.at[pl.ds(0, NREM)], nag_o.at[cid].at[sl])

    return k(m, row3)


def _sc_scatter_t(t16, row3):
    mesh = plsc.VectorSubcoreMesh(core_axis_name="c", subcore_axis_name="s")

    @functools.partial(
        pl.kernel,
        out_type=jax.ShapeDtypeStruct((NC, N, 16), jnp.float32),
        mesh=mesh,
        compiler_params=_UNTILED,
        scratch_types=[
            pltpu.VMEM((NCHK, CH), jnp.int32),
            pltpu.VMEM((CH, 16), jnp.float32),
            pltpu.VMEM((NPT, 16), jnp.float32),
            pltpu.VMEM_SHARED((N, 16), jnp.float32),
        ],
    )
    def k(t_h, row_h, ts_o, ridx, tb, zb16, spt):
        cid = lax.axis_index("c")
        sid = lax.axis_index("s")
        wid = cid * NS + sid

        def zrow16(r, carry):
            zb16[r, :] = jnp.zeros((16,), jnp.float32)
            return carry

        lax.fori_loop(0, NPT, zrow16, 0, unroll=False)
        pltpu.sync_copy(zb16, spt.at[pl.ds(sid * NPT, NPT)])

        @pl.when(sid == NS - 1)
        def _zrem():
            pltpu.sync_copy(zb16.at[pl.ds(0, NREM)],
                            spt.at[pl.ds(NS * NPT, NREM)])

        plsc.subcore_barrier()

        pltpu.sync_copy(row_h.at[wid], ridx)
        base = wid * EPW

        def chunk(c, carry):
            pltpu.sync_copy(t_h.at[pl.ds(base + c * CH, CH)], tb)
            pltpu.sync_copy(tb, spt.at[ridx.at[c]], add=True)
            return carry

        lax.fori_loop(0, NCHK, chunk, 0, unroll=False)
        plsc.subcore_barrier()

        sl = pl.ds(sid * NPT, NPT)
        pltpu.sync_copy(spt.at[sl], zb16)
        pltpu.sync_copy(zb16, ts_o.at[cid].at[sl])

        @pl.when(sid == NS - 1)
        def _wrem():
            sl2 = pl.ds(NS * NPT, NREM)
            pltpu.sync_copy(spt.at[sl2], zb16.at[pl.ds(0, NREM)])
            pltpu.sync_copy(zb16.at[pl.ds(0, NREM)], ts_o.at[cid].at[sl2])

    return k(t16, row3)


def _upd_body(hh, coord, vel, field, velw, nag0, nag1, ts0, ts1,
              nW1, nb1, nW2, nb2, hh_o, coord_o):
    ts = ts0[...] + ts1[...]
    cnt = jnp.maximum(ts[:, 3:4], 1.0)
    coord_o[...] = (coord[...] + ts[:, :3] / cnt
                    + velw[...] * vel[...] + field[...])
    nagg = nag0[...] + nag1[...]
    cat = jnp.concatenate([hh[...], nagg], axis=1)
    h1 = _silu(jnp.dot(cat, nW1[...], preferred_element_type=jnp.float32) + nb1[...])
    hh_o[...] = jnp.dot(h1, nW2[...], preferred_element_type=jnp.float32) + nb2[...]


def _node_update(hh, coord, vel, field, velw, nag0, nag1, ts0, ts1,
                 nW1, nb1, nW2, nb2):
    return pl.pallas_call(
        _upd_body,
        grid=(N // NB,),
        in_specs=[_blk(NB, HID), _blk(NB, 3), _blk(NB, 3), _blk(NB, 3),
                  _blk(NB, 1), _blk(NB, HID), _blk(NB, HID),
                  _blk(NB, 16), _blk(NB, 16),
                  _full((2 * HID, HID)), _full((1, HID)),
                  _full((HID, HID)), _full((1, HID))],
        out_specs=[_blk(NB, HID), _blk(NB, 3)],
        out_shape=[jax.ShapeDtypeStruct((N, HID), jnp.float32),
                   jax.ShapeDtypeStruct((N, 3), jnp.float32)],
    )(hh, coord, vel, field, velw, nag0, nag1, ts0, ts1, nW1, nb1, nW2, nb2)


def kernel(h, x, edges, vel, edge_attr, charges, Wemb, bemb, eW1, eb1, eW2,
           eb2, nW1, nb1, nW2, nb2, cW1, cb1, cW2, vW1, vb1, vW2, vb2, fE,
           fW1, fb1, fW2, fb2, fW3, fb3):
    row3 = edges[0].reshape(NWK, NCHK, CH)
    col3 = edges[1].reshape(NWK, NCHK, CH)
    ch2 = charges.reshape(N, 1)

    field, hh = _node_init(
        x, vel, ch2, h, fE[0:1], fE[1:2],
        fW1, fb1.reshape(1, 32), fW2, fb2.reshape(1, 32),
        fW3, fb3.reshape(1, 3), Wemb, bemb.reshape(1, HID))

    coord = x
    nL = eW1.shape[0]
    for i in range(nL):
        ta, tb, cp, velw = _node_pre(
            hh, coord, eW1[i, :HID], eW1[i, HID:2 * HID],
            eb1[i].reshape(1, HID), vW1[i], vb1[i].reshape(1, HID),
            vW2[i], vb2[i].reshape(1, 1))
        s = _sc_gather(ta, tb, row3, col3)
        cd = _sc_gather_cd(cp, row3, col3)
        m, t16 = _edge_mlp(
            s, cd, edge_attr, eW1[i, HID * 2 + 1:], eW1[i, HID * 2:HID * 2 + 1],
            eW2[i], eb2[i].reshape(1, HID),
            cW1[i], cb1[i].reshape(1, HID), cW2[i])
        nag = _sc_scatter_m(m, row3)
        ts = _sc_scatter_t(t16, row3)
        hh, coord = _node_update(
            hh, coord, vel, field, velw, nag[0], nag[1], ts[0], ts[1],
            nW1[i], nb1[i].reshape(1, HID), nW2[i], nb2[i].reshape(1, HID))
    return coord


# speedup vs baseline: 3.0856x; 1.0179x over previous
import functools

import jax
import jax.numpy as jnp
from jax import lax
from jax.experimental import pallas as pl
from jax.experimental.pallas import tpu as pltpu
from jax.experimental.pallas import tpu_sc as plsc

N = 10000
E = 320000
HID = 128
NC, NS = 2, 16
NWK = NC * NS
EPW = E // NWK
CH = 80
NCHK = EPW // CH
GCH = 64
GNCHK = 158
EPWG = GCH * GNCHK
E2 = NWK * EPWG
NPT = 624
NREM = N - NS * NPT
TRN = 48
NB = 2000
EBK = 2000

_UNTILED = pltpu.CompilerParams(use_tc_tiling_on_sc=False)


def _silu(v):
    return v * jax.nn.sigmoid(v)


def _full(shape):
    return pl.BlockSpec(shape, lambda i: tuple(0 for _ in shape))


def _blk(bs, w):
    return pl.BlockSpec((bs, w), lambda i: (i, 0))


def _init_body(x, vel, ch, h, fE0, fE1, fW1, fb1, fW2, fb2, fW3, fb3,
               Wemb, bemb, field_o, hh_o):
    fec = jnp.where(ch[...] == 0, fE0[...], fE1[...])
    finp = jnp.concatenate([x[...], vel[...], fec], axis=1)
    t = _silu(jnp.dot(finp, fW1[...], preferred_element_type=jnp.float32) + fb1[...])
    t = _silu(jnp.dot(t, fW2[...], preferred_element_type=jnp.float32) + fb2[...])
    field_o[...] = jnp.dot(t, fW3[...], preferred_element_type=jnp.float32) + fb3[...]
    hh_o[...] = jnp.dot(h[...], Wemb[...], preferred_element_type=jnp.float32) + bemb[...]


def _node_init(x, vel, ch, h, fE0, fE1, fW1, fb1, fW2, fb2, fW3, fb3, Wemb, bemb):
    return pl.pallas_call(
        _init_body,
        grid=(N // NB,),
        in_specs=[_blk(NB, 3), _blk(NB, 3), _blk(NB, 1), _blk(NB, 16),
                  _full((1, 16)), _full((1, 16)),
                  _full((22, 32)), _full((1, 32)), _full((32, 32)), _full((1, 32)),
                  _full((32, 3)), _full((1, 3)),
                  _full((16, HID)), _full((1, HID))],
        out_specs=[_blk(NB, 3), _blk(NB, HID)],
        out_shape=[jax.ShapeDtypeStruct((N, 3), jnp.float32),
                   jax.ShapeDtypeStruct((N, HID), jnp.float32)],
    )(x, vel, ch, h, fE0, fE1, fW1, fb1, fW2, fb2, fW3, fb3, Wemb, bemb)


def _pre_body(hh, coord, Wr, Wc, eb1r, vW1, vb1, vW2, vb2,
              tab_o, cpm_o, velw_o):
    b = hh.shape[0]
    ta = jnp.dot(hh[...], Wr[...], preferred_element_type=jnp.float32) + eb1r[...]
    tb = jnp.dot(hh[...], Wc[...], preferred_element_type=jnp.float32)
    tab_o[...] = jnp.stack([ta, tb])
    cp = jnp.concatenate([coord[...], jnp.zeros((b, 13), jnp.float32)], axis=1)
    cpm_o[...] = jnp.stack([cp, -cp])
    v1 = _silu(jnp.dot(hh[...], vW1[...], preferred_element_type=jnp.float32) + vb1[...])
    velw_o[...] = jnp.dot(v1, vW2[...], preferred_element_type=jnp.float32) + vb2[...]


def _node_pre(hh, coord, Wr, Wc, eb1r, vW1, vb1, vW2, vb2):
    return pl.pallas_call(
        _pre_body,
        grid=(N // NB,),
        in_specs=[_blk(NB, HID), _blk(NB, 3),
                  _full((HID, HID)), _full((HID, HID)), _full((1, HID)),
                  _full((HID, HID)), _full((1, HID)), _full((HID, 1)), _full((1, 1))],
        out_specs=[pl.BlockSpec((2, NB, HID), lambda i: (0, i, 0)),
                   pl.BlockSpec((2, NB, 16), lambda i: (0, i, 0)),
                   _blk(NB, 1)],
        out_shape=[jax.ShapeDtypeStruct((2, N, HID), jnp.float32),
                   jax.ShapeDtypeStruct((2, N, 16), jnp.float32),
                   jax.ShapeDtypeStruct((N, 1), jnp.float32)],
    )(hh, coord, Wr, Wc, eb1r, vW1, vb1, vW2, vb2)


def _sc_gather_all(tab2, cpm2, idx3):
    mesh = plsc.VectorSubcoreMesh(core_axis_name="c", subcore_axis_name="s")

    @functools.partial(
        pl.kernel,
        out_type=[jax.ShapeDtypeStruct((E2, HID), jnp.float32),
                  jax.ShapeDtypeStruct((E2, 16), jnp.float32)],
        mesh=mesh,
        compiler_params=_UNTILED,
        scratch_types=[
            pltpu.VMEM((GNCHK, 2 * GCH), jnp.int32),
            pltpu.VMEM((2, 2 * GCH, HID), jnp.float32),
            pltpu.VMEM((2, 2 * GCH, 16), jnp.float32),
            pltpu.VMEM((2, GCH, HID), jnp.float32),
            pltpu.VMEM((2, GCH, 16), jnp.float32),
            pltpu.SemaphoreType.DMA,
            pltpu.SemaphoreType.DMA,
            pltpu.SemaphoreType.DMA,
            pltpu.SemaphoreType.DMA,
        ],
    )
    def k(tab_h, cpm_h, idx_h, s_out, cd_out,
          eidx, gb, cb, sbuf, cdbuf, g0, g1, o0, o1):
        wid = lax.axis_index("c") * NS + lax.axis_index("s")
        pltpu.sync_copy(idx_h.at[wid], eidx)
        base = wid * EPWG
        gsem = (g0, g1)
        osem = (o0, o1)

        def issue(b, c):
            pltpu.async_copy(tab_h.at[eidx.at[c]], gb.at[b], gsem[b])
            pltpu.async_copy(cpm_h.at[eidx.at[c]], cb.at[b], gsem[b])

        def drain_gather(b):
            pltpu.make_async_copy(tab_h.at[eidx.at[0]], gb.at[b], gsem[b]).wait()
            pltpu.make_async_copy(cpm_h.at[eidx.at[0]], cb.at[b], gsem[b]).wait()

        def drain_store(b, c):
            sl = pl.ds(base + c * GCH, GCH)
            pltpu.make_async_copy(sbuf.at[b], s_out.at[sl], osem[b]).wait()
            pltpu.make_async_copy(cdbuf.at[b], cd_out.at[sl], osem[b]).wait()

        issue(0, 0)
        issue(1, 1)

        def pair(g, carry):
            for b in range(2):
                c = 2 * g + b
                drain_gather(b)

                @pl.when(g >= 1)
                def _ds():
                    drain_store(b, c - 2)

                def addrow(r, carry2):
                    for j in range(HID // 16):
                        sl = pl.ds(j * 16, 16)
                        sbuf[b, r, sl] = gb[b, r, sl] + gb[b, GCH + r, sl]
                    cdbuf[b, r, :] = cb[b, r, :] + cb[b, GCH + r, :]
                    return carry2

                lax.fori_loop(0, GCH, addrow, 0, unroll=2)
                sl = pl.ds(base + c * GCH, GCH)
                pltpu.async_copy(sbuf.at[b], s_out.at[sl], osem[b])
                pltpu.async_copy(cdbuf.at[b], cd_out.at[sl], osem[b])

                @pl.when(c + 2 < GNCHK)
                def _ig():
                    issue(b, c + 2)

            return carry

        lax.fori_loop(0, GNCHK // 2, pair, 0, unroll=False)
        drain_store(0, GNCHK - 2)
        drain_store(1, GNCHK - 1)

    return k(tab2, cpm2, idx3)


def _edge_body(s, cdr, ea, Wea, wr, eW2, eb2, cW1, cb1, cW2, m_o, t_o):
    cd = cdr[...]
    radial = jnp.sum(cd * cd, axis=1, keepdims=True)
    pre = (s[...]
           + jnp.dot(ea[...], Wea[...], preferred_element_type=jnp.float32)
           + radial * wr[...])
    m1 = _silu(pre)
    m = _silu(jnp.dot(m1, eW2[...], preferred_element_type=jnp.float32) + eb2[...])
    c1 = _silu(jnp.dot(m, cW1[...], preferred_element_type=jnp.float32) + cb1[...])
    p = jnp.dot(c1, cW2[...], preferred_element_type=jnp.float32)
    lane = lax.broadcasted_iota(jnp.int32, (1, 16), 1)
    t_o[...] = cd * p + jnp.where(lane == 3, 1.0, 0.0)
    m_o[...] = m


def _edge_mlp(s, cd, ea, Wea, wr, eW2, eb2, cW1, cb1, cW2):
    return pl.pallas_call(
        _edge_body,
        grid=(E // EBK,),
        in_specs=[_blk(EBK, HID), _blk(EBK, 16), _blk(EBK, 4),
                  _full((4, HID)), _full((1, HID)),
                  _full((HID, HID)), _full((1, HID)),
                  _full((HID, HID)), _full((1, HID)), _full((HID, 1))],
        out_specs=[_blk(EBK, HID), _blk(EBK, 16)],
        out_shape=[jax.ShapeDtypeStruct((E, HID), jnp.float32),
                   jax.ShapeDtypeStruct((E, 16), jnp.float32)],
    )(s, cd, ea, Wea, wr, eW2, eb2, cW1, cb1, cW2)


def _sc_scatter_m(m, row3):
    mesh = plsc.VectorSubcoreMesh(core_axis_name="c", subcore_axis_name="s")

    @functools.partial(
        pl.kernel,
        out_type=jax.ShapeDtypeStruct((NC, N, HID), jnp.float32),
        mesh=mesh,
        scratch_types=[
            pltpu.VMEM((NCHK, CH), jnp.int32),
            pltpu.VMEM((2, CH, HID), jnp.float32),
            pltpu.VMEM((TRN, HID), jnp.float32),
            pltpu.VMEM_SHARED((N, HID), jnp.float32),
            pltpu.SemaphoreType.DMA,
            pltpu.SemaphoreType.DMA,
        ],
    )
    def k(m_h, row_h, nag_o, ridx, mb, zb, spn, r0, r1):
        cid = lax.axis_index("c")
        sid = lax.axis_index("s")
        wid = cid * NS + sid
        rs = (r0, r1)

        def zrow(r, carry):
            for j in range(HID // 16):
                zb[r, pl.ds(j * 16, 16)] = jnp.zeros((16,), jnp.float32)
            return carry

        lax.fori_loop(0, TRN, zrow, 0, unroll=False)
        for t in range(NPT // TRN):
            pltpu.sync_copy(zb, spn.at[pl.ds(sid * NPT + t * TRN, TRN)])

        @pl.when(sid == NS - 1)
        def _zrem():
            pltpu.sync_copy(zb.at[pl.ds(0, NREM)],
                            spn.at[pl.ds(NS * NPT, NREM)])

        plsc.subcore_barrier()

        pltpu.sync_copy(row_h.at[wid], ridx)
        base = wid * EPW

        def issue(b, c):
            pltpu.async_copy(m_h.at[pl.ds(base + c * CH, CH)], mb.at[b], rs[b])

        def drain(b):
            pltpu.make_async_copy(m_h.at[pl.ds(base, CH)], mb.at[b], rs[b]).wait()

        issue(0, 0)
        issue(1, 1)

        def pairc(g, carry):
            for b in range(2):
                c = 2 * g + b
                drain(b)
                pltpu.sync_copy(mb.at[b], spn.at[ridx.at[c]], add=True)

                @pl.when(c + 2 < NCHK)
                def _ig():
                    issue(b, c + 2)

            return carry

        lax.fori_loop(0, NCHK // 2, pairc, 0, unroll=False)
        drain(0)
        pltpu.sync_copy(mb.at[0], spn.at[ridx.at[NCHK - 1]], add=True)
        plsc.subcore_barrier()

        for t in range(NPT // TRN):
            sl = pl.ds(sid * NPT + t * TRN, TRN)
            pltpu.sync_copy(spn.at[sl], zb)
            pltpu.sync_copy(zb, nag_o.at[cid].at[sl])

        @pl.when(sid == NS - 1)
        def _wrem():
            sl = pl.ds(NS * NPT, NREM)
            pltpu.sync_copy(spn.at[sl], zb.at[pl.ds(0, NREM)])
            pltpu.sync_copy(zb.at[pl.ds(0, NREM)], nag_o.at[cid].at[sl])

    return k(m, row3)


def _sc_scatter_t(t16, row3):
    mesh = plsc.VectorSubcoreMesh(core_axis_name="c", subcore_axis_name="s")

    @functools.partial(
        pl.kernel,
        out_type=jax.ShapeDtypeStruct((NC, N, 16), jnp.float32),
        mesh=mesh,
        compiler_params=_UNTILED,
        scratch_types=[
            pltpu.VMEM((NCHK, CH), jnp.int32),
            pltpu.VMEM((2, CH, 16), jnp.float32),
            pltpu.VMEM((NPT, 16), jnp.float32),
            pltpu.VMEM_SHARED((N, 16), jnp.float32),
            pltpu.SemaphoreType.DMA,
            pltpu.SemaphoreType.DMA,
        ],
    )
    def k(t_h, row_h, ts_o, ridx, tb, zb16, spt, r0, r1):
        cid = lax.axis_index("c")
        sid = lax.axis_index("s")
        wid = cid * NS + sid
        rs = (r0, r1)

        def zrow16(r, carry):
            zb16[r, :] = jnp.zeros((16,), jnp.float32)
            return carry

        lax.fori_loop(0, NPT, zrow16, 0, unroll=False)
        pltpu.sync_copy(zb16, spt.at[pl.ds(sid * NPT, NPT)])

        @pl.when(sid == NS - 1)
        def _zrem():
            pltpu.sync_copy(zb16.at[pl.ds(0, NREM)],
                            spt.at[pl.ds(NS * NPT, NREM)])

        plsc.subcore_barrier()

        pltpu.sync_copy(row_h.at[wid], ridx)
        base = wid * EPW

        def issue(b, c):
            pltpu.async_copy(t_h.at[pl.ds(base + c * CH, CH)], tb.at[b], rs[b])

        def drain(b):
            pltpu.make_async_copy(t_h.at[pl.ds(base, CH)], tb.at[b], rs[b]).wait()

        issue(0, 0)
        issue(1, 1)

        def pairc(g, carry):
            for b in range(2):
                c = 2 * g + b
                drain(b)
                pltpu.sync_copy(tb.at[b], spt.at[ridx.at[c]], add=True)

                @pl.when(c + 2 < NCHK)
                def _ig():
                    issue(b, c + 2)

            return carry

        lax.fori_loop(0, NCHK // 2, pairc, 0, unroll=False)
        drain(0)
        pltpu.sync_copy(tb.at[0], spt.at[ridx.at[NCHK - 1]], add=True)
        plsc.subcore_barrier()

        sl = pl.ds(sid * NPT, NPT)
        pltpu.sync_copy(spt.at[sl], zb16)
        pltpu.sync_copy(zb16, ts_o.at[cid].at[sl])

        @pl.when(sid == NS - 1)
        def _wrem():
            sl2 = pl.ds(NS * NPT, NREM)
            pltpu.sync_copy(spt.at[sl2], zb16.at[pl.ds(0, NREM)])
            pltpu.sync_copy(zb16.at[pl.ds(0, NREM)], ts_o.at[cid].at[sl2])

    return k(t16, row3)


def _upd_body(hh, coord, vel, field, velw, nag0, nag1, ts0, ts1,
              nW1, nb1, nW2, nb2, hh_o, coord_o):
    ts = ts0[...] + ts1[...]
    cnt = jnp.maximum(ts[:, 3:4], 1.0)
    coord_o[...] = (coord[...] + ts[:, :3] / cnt
                    + velw[...] * vel[...] + field[...])
    nagg = nag0[...] + nag1[...]
    cat = jnp.concatenate([hh[...], nagg], axis=1)
    h1 = _silu(jnp.dot(cat, nW1[...], preferred_element_type=jnp.float32) + nb1[...])
    hh_o[...] = jnp.dot(h1, nW2[...], preferred_element_type=jnp.float32) + nb2[...]


def _node_update(hh, coord, vel, field, velw, nag0, nag1, ts0, ts1,
                 nW1, nb1, nW2, nb2):
    return pl.pallas_call(
        _upd_body,
        grid=(N // NB,),
        in_specs=[_blk(NB, HID), _blk(NB, 3), _blk(NB, 3), _blk(NB, 3),
                  _blk(NB, 1), _blk(NB, HID), _blk(NB, HID),
                  _blk(NB, 16), _blk(NB, 16),
                  _full((2 * HID, HID)), _full((1, HID)),
                  _full((HID, HID)), _full((1, HID))],
        out_specs=[_blk(NB, HID), _blk(NB, 3)],
        out_shape=[jax.ShapeDtypeStruct((N, HID), jnp.float32),
                   jax.ShapeDtypeStruct((N, 3), jnp.float32)],
    )(hh, coord, vel, field, velw, nag0, nag1, ts0, ts1, nW1, nb1, nW2, nb2)


def kernel(h, x, edges, vel, edge_attr, charges, Wemb, bemb, eW1, eb1, eW2,
           eb2, nW1, nb1, nW2, nb2, cW1, cb1, cW2, vW1, vb1, vW2, vb2, fE,
           fW1, fb1, fW2, fb2, fW3, fb3):
    row3 = edges[0].reshape(NWK, NCHK, CH)
    pad = jnp.zeros((E2 - E,), jnp.int32)
    idx3 = jnp.concatenate(
        [jnp.concatenate([edges[0], pad]).reshape(NWK, GNCHK, GCH),
         jnp.concatenate([edges[1], pad]).reshape(NWK, GNCHK, GCH) + N],
        axis=2)
    ch2 = charges.reshape(N, 1)

    field, hh = _node_init(
        x, vel, ch2, h, fE[0:1], fE[1:2],
        fW1, fb1.reshape(1, 32), fW2, fb2.reshape(1, 32),
        fW3, fb3.reshape(1, 3), Wemb, bemb.reshape(1, HID))

    coord = x
    nL = eW1.shape[0]
    for i in range(nL):
        tab2, cpm2, velw = _node_pre(
            hh, coord, eW1[i, :HID], eW1[i, HID:2 * HID],
            eb1[i].reshape(1, HID), vW1[i], vb1[i].reshape(1, HID),
            vW2[i], vb2[i].reshape(1, 1))
        s, cd = _sc_gather_all(tab2.reshape(2 * N, HID),
                               cpm2.reshape(2 * N, 16), idx3)
        m, t16 = _edge_mlp(
            s, cd, edge_attr, eW1[i, HID * 2 + 1:], eW1[i, HID * 2:HID * 2 + 1],
            eW2[i], eb2[i].reshape(1, HID),
            cW1[i], cb1[i].reshape(1, HID), cW2[i])
        nag = _sc_scatter_m(m, row3)
        ts = _sc_scatter_t(t16, row3)
        hh, coord = _node_update(
            hh, coord, vel, field, velw, nag[0], nag[1], ts[0], ts[1],
            nW1[i], nb1[i].reshape(1, HID), nW2[i], nb2[i].reshape(1, HID))
    return coord


# speedup vs baseline: 3.1994x; 1.0369x over previous
import functools

import jax
import jax.numpy as jnp
from jax import lax
from jax.experimental import pallas as pl
from jax.experimental.pallas import tpu as pltpu
from jax.experimental.pallas import tpu_sc as plsc

N = 10000
E = 320000
HID = 128
NC, NS = 2, 16
NWK = NC * NS
EPW = E // NWK
CH = 80
NCHK = EPW // CH
GCH = 64
GNCHK = 158
EPWG = GCH * GNCHK
E2 = NWK * EPWG
NPT = 624
NREM = N - NS * NPT
TRN = 48
NB = 2000
EBK = 4000

_UNTILED = pltpu.CompilerParams(use_tc_tiling_on_sc=False)


def _silu(v):
    return v * jax.nn.sigmoid(v)


def _full(shape):
    return pl.BlockSpec(shape, lambda i: tuple(0 for _ in shape))


def _blk(bs, w):
    return pl.BlockSpec((bs, w), lambda i: (i, 0))


def _init_body(x, vel, ch, h, fE0, fE1, fW1, fb1, fW2, fb2, fW3, fb3,
               Wemb, bemb, field_o, hh_o):
    fec = jnp.where(ch[...] == 0, fE0[...], fE1[...])
    finp = jnp.concatenate([x[...], vel[...], fec], axis=1)
    t = _silu(jnp.dot(finp, fW1[...], preferred_element_type=jnp.float32) + fb1[...])
    t = _silu(jnp.dot(t, fW2[...], preferred_element_type=jnp.float32) + fb2[...])
    field_o[...] = jnp.dot(t, fW3[...], preferred_element_type=jnp.float32) + fb3[...]
    hh_o[...] = jnp.dot(h[...], Wemb[...], preferred_element_type=jnp.float32) + bemb[...]


def _node_init(x, vel, ch, h, fE0, fE1, fW1, fb1, fW2, fb2, fW3, fb3, Wemb, bemb):
    return pl.pallas_call(
        _init_body,
        grid=(N // NB,),
        in_specs=[_blk(NB, 3), _blk(NB, 3), _blk(NB, 1), _blk(NB, 16),
                  _full((1, 16)), _full((1, 16)),
                  _full((22, 32)), _full((1, 32)), _full((32, 32)), _full((1, 32)),
                  _full((32, 3)), _full((1, 3)),
                  _full((16, HID)), _full((1, HID))],
        out_specs=[_blk(NB, 3), _blk(NB, HID)],
        out_shape=[jax.ShapeDtypeStruct((N, 3), jnp.float32),
                   jax.ShapeDtypeStruct((N, HID), jnp.float32)],
    )(x, vel, ch, h, fE0, fE1, fW1, fb1, fW2, fb2, fW3, fb3, Wemb, bemb)


def _pre_body(hh, coord, Wr, Wc, eb1r, vW1, vb1, vW2, vb2,
              tab_o, cpm_o, velw_o):
    b = hh.shape[0]
    ta = jnp.dot(hh[...], Wr[...], preferred_element_type=jnp.float32) + eb1r[...]
    tb = jnp.dot(hh[...], Wc[...], preferred_element_type=jnp.float32)
    tab_o[...] = jnp.stack([ta, tb])
    cp = jnp.concatenate([coord[...], jnp.zeros((b, 13), jnp.float32)], axis=1)
    cpm_o[...] = jnp.stack([cp, -cp])
    v1 = _silu(jnp.dot(hh[...], vW1[...], preferred_element_type=jnp.float32) + vb1[...])
    velw_o[...] = jnp.dot(v1, vW2[...], preferred_element_type=jnp.float32) + vb2[...]


def _node_pre(hh, coord, Wr, Wc, eb1r, vW1, vb1, vW2, vb2):
    return pl.pallas_call(
        _pre_body,
        grid=(N // NB,),
        in_specs=[_blk(NB, HID), _blk(NB, 3),
                  _full((HID, HID)), _full((HID, HID)), _full((1, HID)),
                  _full((HID, HID)), _full((1, HID)), _full((HID, 1)), _full((1, 1))],
        out_specs=[pl.BlockSpec((2, NB, HID), lambda i: (0, i, 0)),
                   pl.BlockSpec((2, NB, 16), lambda i: (0, i, 0)),
                   _blk(NB, 1)],
        out_shape=[jax.ShapeDtypeStruct((2, N, HID), jnp.float32),
                   jax.ShapeDtypeStruct((2, N, 16), jnp.float32),
                   jax.ShapeDtypeStruct((N, 1), jnp.float32)],
    )(hh, coord, Wr, Wc, eb1r, vW1, vb1, vW2, vb2)


def _sc_gather_all(tab2, cpm2, idx3):
    mesh = plsc.VectorSubcoreMesh(core_axis_name="c", subcore_axis_name="s")

    @functools.partial(
        pl.kernel,
        out_type=[jax.ShapeDtypeStruct((E2, HID), jnp.float32),
                  jax.ShapeDtypeStruct((E2, 16), jnp.float32)],
        mesh=mesh,
        compiler_params=_UNTILED,
        scratch_types=[
            pltpu.VMEM((GNCHK, 2 * GCH), jnp.int32),
            pltpu.VMEM((2, 2 * GCH, HID), jnp.float32),
            pltpu.VMEM((2, 2 * GCH, 16), jnp.float32),
            pltpu.VMEM((2, GCH, HID), jnp.float32),
            pltpu.VMEM((2, GCH, 16), jnp.float32),
            pltpu.SemaphoreType.DMA,
            pltpu.SemaphoreType.DMA,
            pltpu.SemaphoreType.DMA,
            pltpu.SemaphoreType.DMA,
        ],
    )
    def k(tab_h, cpm_h, idx_h, s_out, cd_out,
          eidx, gb, cb, sbuf, cdbuf, g0, g1, o0, o1):
        wid = lax.axis_index("c") * NS + lax.axis_index("s")
        pltpu.sync_copy(idx_h.at[wid], eidx)
        base = wid * EPWG
        gsem = (g0, g1)
        osem = (o0, o1)

        def issue(b, c):
            pltpu.async_copy(tab_h.at[eidx.at[c]], gb.at[b], gsem[b])
            pltpu.async_copy(cpm_h.at[eidx.at[c]], cb.at[b], gsem[b])

        def drain_gather(b):
            pltpu.make_async_copy(tab_h.at[eidx.at[0]], gb.at[b], gsem[b]).wait()
            pltpu.make_async_copy(cpm_h.at[eidx.at[0]], cb.at[b], gsem[b]).wait()

        def drain_store(b, c):
            sl = pl.ds(base + c * GCH, GCH)
            pltpu.make_async_copy(sbuf.at[b], s_out.at[sl], osem[b]).wait()
            pltpu.make_async_copy(cdbuf.at[b], cd_out.at[sl], osem[b]).wait()

        issue(0, 0)
        issue(1, 1)

        def pair(g, carry):
            for b in range(2):
                c = 2 * g + b
                drain_gather(b)

                @pl.when(g >= 1)
                def _ds():
                    drain_store(b, c - 2)

                def addrow(r, carry2):
                    for j in range(HID // 16):
                        sl = pl.ds(j * 16, 16)
                        sbuf[b, r, sl] = gb[b, r, sl] + gb[b, GCH + r, sl]
                    cdbuf[b, r, :] = cb[b, r, :] + cb[b, GCH + r, :]
                    return carry2

                lax.fori_loop(0, GCH, addrow, 0, unroll=2)
                sl = pl.ds(base + c * GCH, GCH)
                pltpu.async_copy(sbuf.at[b], s_out.at[sl], osem[b])
                pltpu.async_copy(cdbuf.at[b], cd_out.at[sl], osem[b])

                @pl.when(c + 2 < GNCHK)
                def _ig():
                    issue(b, c + 2)

            return carry

        lax.fori_loop(0, GNCHK // 2, pair, 0, unroll=False)
        drain_store(0, GNCHK - 2)
        drain_store(1, GNCHK - 1)

    return k(tab2, cpm2, idx3)


def _bmm(a, b):
    return jnp.dot(a.astype(jnp.bfloat16), b.astype(jnp.bfloat16),
                   preferred_element_type=jnp.float32)


def _edge_body(s, cdr, ea, Wea, wr, eW2, eb2, cW1, cb1, cW2, m_o, t_o):
    cd = cdr[...]
    radial = jnp.sum(cd * cd, axis=1, keepdims=True)
    pre = (s[...]
           + jnp.dot(ea[...], Wea[...], preferred_element_type=jnp.float32)
           + radial * wr[...])
    m1 = _silu(pre)
    m = _silu(_bmm(m1, eW2[...]) + eb2[...])
    c1 = _silu(_bmm(m, cW1[...]) + cb1[...])
    p = _bmm(c1, cW2[...])
    lane = lax.broadcasted_iota(jnp.int32, (1, 16), 1)
    t_o[...] = cd * p + jnp.where(lane == 3, 1.0, 0.0)
    m_o[...] = m


def _edge_mlp(s, cd, ea, Wea, wr, eW2, eb2, cW1, cb1, cW2):
    return pl.pallas_call(
        _edge_body,
        grid=(E // EBK,),
        in_specs=[_blk(EBK, HID), _blk(EBK, 16), _blk(EBK, 4),
                  _full((4, HID)), _full((1, HID)),
                  _full((HID, HID)), _full((1, HID)),
                  _full((HID, HID)), _full((1, HID)), _full((HID, 1))],
        out_specs=[_blk(EBK, HID), _blk(EBK, 16)],
        out_shape=[jax.ShapeDtypeStruct((E, HID), jnp.float32),
                   jax.ShapeDtypeStruct((E, 16), jnp.float32)],
    )(s, cd, ea, Wea, wr, eW2, eb2, cW1, cb1, cW2)


def _sc_scatter_m(m, row3):
    mesh = plsc.VectorSubcoreMesh(core_axis_name="c", subcore_axis_name="s")

    @functools.partial(
        pl.kernel,
        out_type=jax.ShapeDtypeStruct((NC, N, HID), jnp.float32),
        mesh=mesh,
        scratch_types=[
            pltpu.VMEM((NCHK, CH), jnp.int32),
            pltpu.VMEM((2, CH, HID), jnp.float32),
            pltpu.VMEM((TRN, HID), jnp.float32),
            pltpu.VMEM_SHARED((N, HID), jnp.float32),
            pltpu.SemaphoreType.DMA,
            pltpu.SemaphoreType.DMA,
        ],
    )
    def k(m_h, row_h, nag_o, ridx, mb, zb, spn, r0, r1):
        cid = lax.axis_index("c")
        sid = lax.axis_index("s")
        wid = cid * NS + sid
        rs = (r0, r1)

        def zrow(r, carry):
            for j in range(HID // 16):
                zb[r, pl.ds(j * 16, 16)] = jnp.zeros((16,), jnp.float32)
            return carry

        lax.fori_loop(0, TRN, zrow, 0, unroll=False)
        for t in range(NPT // TRN):
            pltpu.sync_copy(zb, spn.at[pl.ds(sid * NPT + t * TRN, TRN)])

        @pl.when(sid == NS - 1)
        def _zrem():
            pltpu.sync_copy(zb.at[pl.ds(0, NREM)],
                            spn.at[pl.ds(NS * NPT, NREM)])

        plsc.subcore_barrier()

        pltpu.sync_copy(row_h.at[wid], ridx)
        base = wid * EPW

        def issue(b, c):
            pltpu.async_copy(m_h.at[pl.ds(base + c * CH, CH)], mb.at[b], rs[b])

        def drain(b):
            pltpu.make_async_copy(m_h.at[pl.ds(base, CH)], mb.at[b], rs[b]).wait()

        issue(0, 0)
        issue(1, 1)

        def pairc(g, carry):
            for b in range(2):
                c = 2 * g + b
                drain(b)
                pltpu.sync_copy(mb.at[b], spn.at[ridx.at[c]], add=True)

                @pl.when(c + 2 < NCHK)
                def _ig():
                    issue(b, c + 2)

            return carry

        lax.fori_loop(0, NCHK // 2, pairc, 0, unroll=False)
        drain(0)
        pltpu.sync_copy(mb.at[0], spn.at[ridx.at[NCHK - 1]], add=True)
        plsc.subcore_barrier()

        for t in range(NPT // TRN):
            sl = pl.ds(sid * NPT + t * TRN, TRN)
            pltpu.sync_copy(spn.at[sl], zb)
            pltpu.sync_copy(zb, nag_o.at[cid].at[sl])

        @pl.when(sid == NS - 1)
        def _wrem():
            sl = pl.ds(NS * NPT, NREM)
            pltpu.sync_copy(spn.at[sl], zb.at[pl.ds(0, NREM)])
            pltpu.sync_copy(zb.at[pl.ds(0, NREM)], nag_o.at[cid].at[sl])

    return k(m, row3)


def _sc_scatter_t(t16, row3):
    mesh = plsc.VectorSubcoreMesh(core_axis_name="c", subcore_axis_name="s")

    @functools.partial(
        pl.kernel,
        out_type=jax.ShapeDtypeStruct((NC, N, 16), jnp.float32),
        mesh=mesh,
        compiler_params=_UNTILED,
        scratch_types=[
            pltpu.VMEM((NCHK, CH), jnp.int32),
            pltpu.VMEM((2, CH, 16), jnp.float32),
            pltpu.VMEM((NPT, 16), jnp.float32),
            pltpu.VMEM_SHARED((N, 16), jnp.float32),
            pltpu.SemaphoreType.DMA,
            pltpu.SemaphoreType.DMA,
        ],
    )
    def k(t_h, row_h, ts_o, ridx, tb, zb16, spt, r0, r1):
        cid = lax.axis_index("c")
        sid = lax.axis_index("s")
        wid = cid * NS + sid
        rs = (r0, r1)

        def zrow16(r, carry):
            zb16[r, :] = jnp.zeros((16,), jnp.float32)
            return carry

        lax.fori_loop(0, NPT, zrow16, 0, unroll=False)
        pltpu.sync_copy(zb16, spt.at[pl.ds(sid * NPT, NPT)])

        @pl.when(sid == NS - 1)
        def _zrem():
            pltpu.sync_copy(zb16.at[pl.ds(0, NREM)],
                            spt.at[pl.ds(NS * NPT, NREM)])

        plsc.subcore_barrier()

        pltpu.sync_copy(row_h.at[wid], ridx)
        base = wid * EPW

        def issue(b, c):
            pltpu.async_copy(t_h.at[pl.ds(base + c * CH, CH)], tb.at[b], rs[b])

        def drain(b):
            pltpu.make_async_copy(t_h.at[pl.ds(base, CH)], tb.at[b], rs[b]).wait()

        issue(0, 0)
        issue(1, 1)

        def pairc(g, carry):
            for b in range(2):
                c = 2 * g + b
                drain(b)
                pltpu.sync_copy(tb.at[b], spt.at[ridx.at[c]], add=True)

                @pl.when(c + 2 < NCHK)
                def _ig():
                    issue(b, c + 2)

            return carry

        lax.fori_loop(0, NCHK // 2, pairc, 0, unroll=False)
        drain(0)
        pltpu.sync_copy(tb.at[0], spt.at[ridx.at[NCHK - 1]], add=True)
        plsc.subcore_barrier()

        sl = pl.ds(sid * NPT, NPT)
        pltpu.sync_copy(spt.at[sl], zb16)
        pltpu.sync_copy(zb16, ts_o.at[cid].at[sl])

        @pl.when(sid == NS - 1)
        def _wrem():
            sl2 = pl.ds(NS * NPT, NREM)
            pltpu.sync_copy(spt.at[sl2], zb16.at[pl.ds(0, NREM)])
            pltpu.sync_copy(zb16.at[pl.ds(0, NREM)], ts_o.at[cid].at[sl2])

    return k(t16, row3)


def _upd_body(hh, coord, vel, field, velw, nag0, nag1, ts0, ts1,
              nW1, nb1, nW2, nb2, hh_o, coord_o):
    ts = ts0[...] + ts1[...]
    cnt = jnp.maximum(ts[:, 3:4], 1.0)
    coord_o[...] = (coord[...] + ts[:, :3] / cnt
                    + velw[...] * vel[...] + field[...])
    nagg = nag0[...] + nag1[...]
    cat = jnp.concatenate([hh[...], nagg], axis=1)
    h1 = _silu(jnp.dot(cat, nW1[...], preferred_element_type=jnp.float32) + nb1[...])
    hh_o[...] = jnp.dot(h1, nW2[...], preferred_element_type=jnp.float32) + nb2[...]


def _node_update(hh, coord, vel, field, velw, nag0, nag1, ts0, ts1,
                 nW1, nb1, nW2, nb2):
    return pl.pallas_call(
        _upd_body,
        grid=(N // NB,),
        in_specs=[_blk(NB, HID), _blk(NB, 3), _blk(NB, 3), _blk(NB, 3),
                  _blk(NB, 1), _blk(NB, HID), _blk(NB, HID),
                  _blk(NB, 16), _blk(NB, 16),
                  _full((2 * HID, HID)), _full((1, HID)),
                  _full((HID, HID)), _full((1, HID))],
        out_specs=[_blk(NB, HID), _blk(NB, 3)],
        out_shape=[jax.ShapeDtypeStruct((N, HID), jnp.float32),
                   jax.ShapeDtypeStruct((N, 3), jnp.float32)],
    )(hh, coord, vel, field, velw, nag0, nag1, ts0, ts1, nW1, nb1, nW2, nb2)


def kernel(h, x, edges, vel, edge_attr, charges, Wemb, bemb, eW1, eb1, eW2,
           eb2, nW1, nb1, nW2, nb2, cW1, cb1, cW2, vW1, vb1, vW2, vb2, fE,
           fW1, fb1, fW2, fb2, fW3, fb3):
    row3 = edges[0].reshape(NWK, NCHK, CH)
    pad = jnp.zeros((E2 - E,), jnp.int32)
    idx3 = jnp.concatenate(
        [jnp.concatenate([edges[0], pad]).reshape(NWK, GNCHK, GCH),
         jnp.concatenate([edges[1], pad]).reshape(NWK, GNCHK, GCH) + N],
        axis=2)
    ch2 = charges.reshape(N, 1)

    field, hh = _node_init(
        x, vel, ch2, h, fE[0:1], fE[1:2],
        fW1, fb1.reshape(1, 32), fW2, fb2.reshape(1, 32),
        fW3, fb3.reshape(1, 3), Wemb, bemb.reshape(1, HID))

    coord = x
    nL = eW1.shape[0]
    for i in range(nL):
        tab2, cpm2, velw = _node_pre(
            hh, coord, eW1[i, :HID], eW1[i, HID:2 * HID],
            eb1[i].reshape(1, HID), vW1[i], vb1[i].reshape(1, HID),
            vW2[i], vb2[i].reshape(1, 1))
        s, cd = _sc_gather_all(tab2.reshape(2 * N, HID),
                               cpm2.reshape(2 * N, 16), idx3)
        m, t16 = _edge_mlp(
            s, cd, edge_attr, eW1[i, HID * 2 + 1:], eW1[i, HID * 2:HID * 2 + 1],
            eW2[i], eb2[i].reshape(1, HID),
            cW1[i], cb1[i].reshape(1, HID), cW2[i])
        nag = _sc_scatter_m(m, row3)
        ts = _sc_scatter_t(t16, row3)
        hh, coord = _node_update(
            hh, coord, vel, field, velw, nag[0], nag[1], ts[0], ts[1],
            nW1[i], nb1[i].reshape(1, HID), nW2[i], nb2[i].reshape(1, HID))
    return coord
